# baseline (speedup 1.0000x reference)
import jax
import jax.numpy as jnp
from jax import lax
from jax.experimental import pallas as pl
from jax.experimental.pallas import tpu as pltpu
from jax.experimental.pallas import tpu_sc as plsc

B = 8
N = 20000
K = 200
C = 5
L = 16
KPAD = 208
EPAD = KPAD * 4
HALF = KPAD // 2
BN = B * N
ZSENT = 6 * BN
PHA = 4000
STEP_ROWS = 2 * L
BLK_STEPS = 25
PHA_BLKS = PHA // (STEP_ROWS * BLK_STEPS)
PHB_BLKS = (N - PHA) // (STEP_ROWS * BLK_STEPS)


def _body(cols_hbm, out_hbm,
          cls_v, idx_v, idx2_v, rows_v, cnt_ref, sem):
    cid = lax.axis_index("c")
    sid = lax.axis_index("s")
    w = sid * 2 + cid
    b = w % B
    iota = lax.iota(jnp.int32, L)

    pltpu.sync_copy(cols_hbm.at[pl.ds(b * N, PHA)], cls_v.at[pl.ds(0, PHA)])

    def scan_blocks(start_blk, nblks, classes, offs):
        def outer(blk, carry):
            live = cnt_ref[0] < K
            if len(classes) > 1:
                live = jnp.logical_or(live, cnt_ref[1] < K)

            @pl.when(live)
            def _():
                def inner(j, cnts):
                    r0 = (blk * BLK_STEPS + j) * STEP_ROWS
                    v0 = cls_v[pl.ds(r0, L)]
                    v1 = cls_v[pl.ds(r0 + L, L)]
                    new = []
                    for ci, (cc, off) in enumerate(zip(classes, offs)):
                        cnt = cnts[ci]
                        m0 = v0 == float(cc)
                        m1 = v1 == float(cc)
                        cs0 = plsc.cumsum(jnp.where(m0, 1, 0).astype(jnp.int32))
                        cs1 = plsc.cumsum(jnp.where(m1, 1, 0).astype(jnp.int32))
                        s0 = cs0[L - 1]
                        pos0 = cnt + cs0 - 1
                        pos1 = cnt + s0 + cs1 - 1
                        plsc.store_scatter(
                            idx_v, [pos0 + off], r0 + iota,
                            mask=jnp.logical_and(m0, pos0 < K))
                        plsc.store_scatter(
                            idx_v, [pos1 + off], r0 + L + iota,
                            mask=jnp.logical_and(m1, pos1 < K))
                        new.append(cnt + s0 + cs1[L - 1])
                    return tuple(new)

                res = lax.fori_loop(
                    0, BLK_STEPS, inner,
                    tuple(cnt_ref[ci] for ci in range(len(classes))))
                for ci in range(len(classes)):
                    cnt_ref[ci] = res[ci]
            return carry

        lax.fori_loop(start_blk, start_blk + nblks, outer, jnp.int32(0))

    def finish_task(cc, off, ci):
        colsel = iota & 3
        rowsel = iota >> 2
        pvec = (2 + colsel) * BN + b * N
        GRP = 13
        for qg in range(0, EPAD // L, GRP):
            rows = [plsc.load_gather(idx_v, [rowsel + (off + q * 4)])
                    for q in range(qg, qg + GRP)]
            for q, row in zip(range(qg, qg + GRP), rows):
                idx2_v[pl.ds(q * L, L)] = jnp.minimum(row + pvec, ZSENT)

        cps = []
        for h in range(EPAD // HALF):
            cps.append(pltpu.async_copy(
                cols_hbm.at[idx2_v.at[pl.ds(h * HALF, HALF)]],
                rows_v.at[pl.ds(h * HALF, HALF)], sem))
        for cp in cps:
            cp.wait()

        t = (cc - 1) * B + b
        pltpu.sync_copy(
            rows_v.at[pl.ds(0, K * 4)], out_hbm.at[pl.ds(t * K * 4, K * 4)])

    def run(classes):
        offs = tuple(ci * KPAD for ci in range(len(classes)))
        for kk in range(len(classes) * KPAD // L):
            idx_v[pl.ds(kk * L, L)] = jnp.full((L,), ZSENT, jnp.int32)
        for ci in range(len(classes)):
            cnt_ref[ci] = jnp.int32(0)

        scan_blocks(0, PHA_BLKS, classes, offs)
        live = cnt_ref[0] < K
        if len(classes) > 1:
            live = jnp.logical_or(live, cnt_ref[1] < K)

        @pl.when(live)
        def _():
            pltpu.sync_copy(
                cols_hbm.at[pl.ds(b * N + PHA, N - PHA)],
                cls_v.at[pl.ds(PHA, N - PHA)])
            scan_blocks(PHA_BLKS, PHB_BLKS, classes, offs)

        for ci, (cc, off) in enumerate(zip(classes, offs)):
            finish_task(cc, off, ci)

    @pl.when(w < B)
    def _():
        run((1, 5))

    for g in (1, 2, 3):
        @pl.when(w // B == g)
        def _(g=g):
            run((g + 1,))


def kernel(rois_all):
    cols_flat = jnp.concatenate(
        [jnp.moveaxis(rois_all, 2, 0).reshape(6 * BN),
         jnp.zeros((8,), jnp.float32)])

    mesh = plsc.VectorSubcoreMesh(
        core_axis_name="c", subcore_axis_name="s", num_cores=2, num_subcores=16)
    outs = pl.kernel(
        _body,
        out_type=jax.ShapeDtypeStruct((C * B * K * 4,), jnp.float32),
        mesh=mesh,
        compiler_params=pltpu.CompilerParams(needs_layout_passes=False),
        scratch_types=[
            pltpu.VMEM((N,), jnp.float32),
            pltpu.VMEM((2 * KPAD,), jnp.int32),
            pltpu.VMEM((EPAD,), jnp.int32),
            pltpu.VMEM((EPAD,), jnp.float32),
            pltpu.SMEM((2,), jnp.int32),
            pltpu.SemaphoreType.DMA,
        ],
    )(cols_flat)

    s = B * K * 4
    return tuple(
        outs[i * s:(i + 1) * s].reshape(B, K, 4) for i in range(C))


# speedup vs baseline: 1.1396x; 1.1396x over previous
import jax
import jax.numpy as jnp
from jax import lax
from jax.experimental import pallas as pl
from jax.experimental.pallas import tpu as pltpu
from jax.experimental.pallas import tpu_sc as plsc

B = 8
N = 20000
K = 200
C = 5
L = 16
KPAD = 208
EPAD = KPAD * 4
HALF = KPAD // 2
BN = B * N
PHA = 4000
STEP_ROWS = 2 * L
BLK_STEPS = 25
PHA_BLKS = PHA // (STEP_ROWS * BLK_STEPS)
PHB_BLKS = (N - PHA) // (STEP_ROWS * BLK_STEPS)


def _body(cols_hbm, out_hbm,
          cls_v, idx_v, idx2_v, rows_v, cnt_ref, sem):
    cid = lax.axis_index("c")
    sid = lax.axis_index("s")
    w = sid * 2 + cid
    b = w % B
    iota = lax.iota(jnp.int32, L)

    pltpu.sync_copy(cols_hbm.at[pl.ds(b * N, PHA)], cls_v.at[pl.ds(0, PHA)])

    def scan_blocks(start_blk, nblks, classes, offs):
        def outer(blk, carry):
            live = cnt_ref[0] < K
            if len(classes) > 1:
                live = jnp.logical_or(live, cnt_ref[1] < K)

            @pl.when(live)
            def _():
                def inner(j, cnts):
                    r0 = (blk * BLK_STEPS + j) * STEP_ROWS
                    v0 = cls_v[pl.ds(r0, L)]
                    v1 = cls_v[pl.ds(r0 + L, L)]
                    new = []
                    for ci, (cc, off) in enumerate(zip(classes, offs)):
                        cnt = cnts[ci]
                        m0 = v0 == float(cc)
                        m1 = v1 == float(cc)
                        cs0 = plsc.cumsum(jnp.where(m0, 1, 0).astype(jnp.int32))
                        cs1 = plsc.cumsum(jnp.where(m1, 1, 0).astype(jnp.int32))
                        s0 = cs0[L - 1]
                        pos0 = cnt + cs0 - 1
                        pos1 = cnt + s0 + cs1 - 1
                        plsc.store_scatter(
                            idx_v, [pos0 + off], r0 + iota,
                            mask=jnp.logical_and(m0, pos0 < K))
                        plsc.store_scatter(
                            idx_v, [pos1 + off], r0 + L + iota,
                            mask=jnp.logical_and(m1, pos1 < K))
                        new.append(cnt + s0 + cs1[L - 1])
                    return tuple(new)

                res = lax.fori_loop(
                    0, BLK_STEPS, inner,
                    tuple(cnt_ref[ci] for ci in range(len(classes))))
                for ci in range(len(classes)):
                    cnt_ref[ci] = res[ci]
            return carry

        lax.fori_loop(start_blk, start_blk + nblks, outer, jnp.int32(0))

    colsel = iota & 3
    rowsel = iota >> 2
    GRP = 13

    def expand_and_fire(off, ci):
        pvec = (2 + colsel) * BN + b * N
        eoff = ci * EPAD
        for qg in range(0, EPAD // L, GRP):
            rows = [plsc.load_gather(idx_v, [rowsel + (off + q * 4)])
                    for q in range(qg, qg + GRP)]
            for q, row in zip(range(qg, qg + GRP), rows):
                idx2_v[pl.ds(eoff + q * L, L)] = row + pvec
        return [pltpu.async_copy(
            cols_hbm.at[idx2_v.at[pl.ds(eoff + h * HALF, HALF)]],
            rows_v.at[pl.ds(eoff + h * HALF, HALF)], sem)
            for h in range(EPAD // HALF)]

    def write_task(cc, ci):
        cnt4 = cnt_ref[ci] * 4
        eoff = ci * EPAD
        for qg in range(0, EPAD // L, GRP):
            vals = [rows_v[pl.ds(eoff + q * L, L)]
                    for q in range(qg, qg + GRP)]
            for q, val in zip(range(qg, qg + GRP), vals):
                e = q * L + iota
                rows_v[pl.ds(eoff + q * L, L)] = jnp.where(e < cnt4, val, 0.0)
        t = (cc - 1) * B + b
        pltpu.sync_copy(
            rows_v.at[pl.ds(eoff, K * 4)], out_hbm.at[pl.ds(t * K * 4, K * 4)])

    def run(classes):
        offs = tuple(ci * KPAD for ci in range(len(classes)))
        for kk in range(len(classes) * KPAD // L):
            idx_v[pl.ds(kk * L, L)] = jnp.zeros((L,), jnp.int32)
        for ci in range(len(classes)):
            cnt_ref[ci] = jnp.int32(0)

        scan_blocks(0, PHA_BLKS, classes, offs)
        live = cnt_ref[0] < K
        if len(classes) > 1:
            live = jnp.logical_or(live, cnt_ref[1] < K)

        @pl.when(live)
        def _():
            pltpu.sync_copy(
                cols_hbm.at[pl.ds(b * N + PHA, N - PHA)],
                cls_v.at[pl.ds(PHA, N - PHA)])
            scan_blocks(PHA_BLKS, PHB_BLKS, classes, offs)

        cps = []
        for ci, off in enumerate(offs):
            cps.extend(expand_and_fire(off, ci))
        for cp in cps:
            cp.wait()
        for ci, cc in enumerate(classes):
            write_task(cc, ci)

    @pl.when(w < B)
    def _():
        run((1, 5))

    for g in (1, 2, 3):
        @pl.when(w // B == g)
        def _(g=g):
            run((g + 1,))


def kernel(rois_all):
    cols_flat = jnp.moveaxis(rois_all, 2, 0).reshape(6 * BN)

    mesh = plsc.VectorSubcoreMesh(
        core_axis_name="c", subcore_axis_name="s", num_cores=2, num_subcores=16)
    outs = pl.kernel(
        _body,
        out_type=jax.ShapeDtypeStruct((C * B * K * 4,), jnp.float32),
        mesh=mesh,
        compiler_params=pltpu.CompilerParams(needs_layout_passes=False),
        scratch_types=[
            pltpu.VMEM((N,), jnp.float32),
            pltpu.VMEM((2 * KPAD,), jnp.int32),
            pltpu.VMEM((2 * EPAD,), jnp.int32),
            pltpu.VMEM((2 * EPAD,), jnp.float32),
            pltpu.SMEM((2,), jnp.int32),
            pltpu.SemaphoreType.DMA,
        ],
    )(cols_flat)

    s = B * K * 4
    return tuple(
        outs[i * s:(i + 1) * s].reshape(B, K, 4) for i in range(C))
